# baseline (speedup 1.0000x reference)
import functools

import jax
import jax.numpy as jnp
from jax import lax
from jax.experimental import pallas as pl
from jax.experimental.pallas import tpu as pltpu
from jax.experimental.pallas import tpu_sc as plsc

ROWS = 16384
COLS = 200
NUM_CORES = 2
NUM_SUBCORES = 16
NW = NUM_CORES * NUM_SUBCORES
ROWS_PER_W = ROWS // NW
BLK_ROWS = 64
NBLK = ROWS_PER_W // BLK_ROWS
LANES = 16
ROW_OFFS = tuple(j * LANES for j in range(COLS // LANES)) + (COLS - LANES,)

_mesh = plsc.VectorSubcoreMesh(core_axis_name="c", subcore_axis_name="s")


@functools.partial(
    pl.kernel,
    mesh=_mesh,
    out_type=jax.ShapeDtypeStruct((ROWS, COLS), jnp.float32),
    compiler_params=pltpu.CompilerParams(
        needs_layout_passes=False,
        disable_bounds_checks=True,
        disable_semaphore_checks=True,
        skip_device_barrier=True,
    ),
    scratch_types=[
        pltpu.VMEM((64,), jnp.float32),
        pltpu.VMEM((64,), jnp.float32),
        pltpu.VMEM((BLK_ROWS, COLS), jnp.int32),
        pltpu.VMEM((BLK_ROWS, COLS), jnp.int32),
        pltpu.VMEM((BLK_ROWS, COLS), jnp.float32),
        pltpu.VMEM((BLK_ROWS, COLS), jnp.float32),
        pltpu.SemaphoreType.DMA,
        pltpu.SemaphoreType.DMA,
        pltpu.SemaphoreType.DMA,
        pltpu.SemaphoreType.DMA,
    ],
)
def _sc_lookup(ids_hbm, hop_hbm, cut_hbm, out_hbm,
               table_v, cut_v, ids_v0, ids_v1, out_v0, out_v1,
               in_sem0, in_sem1, out_sem0, out_sem1):
    wid = lax.axis_index("s") * NUM_CORES + lax.axis_index("c")
    base = wid * ROWS_PER_W
    ids_bufs = (ids_v0, ids_v1)
    out_bufs = (out_v0, out_v1)
    in_sems = (in_sem0, in_sem1)
    out_sems = (out_sem0, out_sem1)

    pltpu.sync_copy(hop_hbm, table_v)
    pltpu.sync_copy(cut_hbm, cut_v)
    for i in range(64 // LANES):
        sl = pl.ds(i * LANES, LANES)
        table_v[sl] = table_v[sl] + cut_v[sl]

    def start_in(b):
        r0 = base + b * BLK_ROWS
        return pltpu.async_copy(
            ids_hbm.at[pl.ds(r0, BLK_ROWS)], ids_bufs[b % 2], in_sems[b % 2])

    def start_out(b):
        r0 = base + b * BLK_ROWS
        return pltpu.async_copy(
            out_bufs[b % 2], out_hbm.at[pl.ds(r0, BLK_ROWS)], out_sems[b % 2])

    in_dmas = {0: start_in(0)}
    out_dmas = {}
    for b in range(NBLK):
        if b + 1 < NBLK:
            in_dmas[b + 1] = start_in(b + 1)
        in_dmas[b].wait()
        if b >= 2:
            out_dmas[b - 2].wait()

        ids_b = ids_bufs[b % 2]
        out_b = out_bufs[b % 2]

        @plsc.parallel_loop(0, BLK_ROWS, 1, unroll=4)
        def gather_body(r):
            for off in ROW_OFFS:
                sl = pl.ds(off, LANES)
                out_b[r, sl] = plsc.load_gather(table_v, [ids_b[r, sl]])

        out_dmas[b] = start_out(b)

    out_dmas[NBLK - 2].wait()
    out_dmas[NBLK - 1].wait()


def kernel(ids_mat, hop_table, cut_off_table):
    return _sc_lookup(ids_mat, hop_table, cut_off_table)


# speedup vs baseline: 1.8407x; 1.8407x over previous
import functools

import jax
import jax.numpy as jnp
from jax import lax
from jax.experimental import pallas as pl
from jax.experimental.pallas import tpu as pltpu
from jax.experimental.pallas import tpu_sc as plsc

ROWS = 200
COLS = 16384
NUM_CORES = 2
NUM_SUBCORES = 16
NW = NUM_CORES * NUM_SUBCORES
COLS_PER_W = COLS // NW
BLK_COLS = 128
NBLK = COLS_PER_W // BLK_COLS
LANES = 16
VECS_PER_ROW = BLK_COLS // LANES

_mesh = plsc.VectorSubcoreMesh(core_axis_name="c", subcore_axis_name="s")


@functools.partial(
    pl.kernel,
    mesh=_mesh,
    out_type=jax.ShapeDtypeStruct((ROWS, COLS), jnp.float32),
    compiler_params=pltpu.CompilerParams(
        needs_layout_passes=False,
        disable_bounds_checks=True,
        disable_semaphore_checks=True,
        skip_device_barrier=True,
    ),
    scratch_types=[
        pltpu.VMEM((64,), jnp.float32),
        pltpu.VMEM((64,), jnp.float32),
        pltpu.VMEM((ROWS, BLK_COLS), jnp.int32),
        pltpu.VMEM((ROWS, BLK_COLS), jnp.int32),
        pltpu.VMEM((ROWS, BLK_COLS), jnp.float32),
        pltpu.VMEM((ROWS, BLK_COLS), jnp.float32),
        pltpu.SemaphoreType.DMA,
        pltpu.SemaphoreType.DMA,
        pltpu.SemaphoreType.DMA,
        pltpu.SemaphoreType.DMA,
    ],
)
def _sc_lookup(ids_hbm, hop_hbm, cut_hbm, out_hbm,
               table_v, cut_v, ids_v0, ids_v1, out_v0, out_v1,
               in_sem0, in_sem1, out_sem0, out_sem1):
    wid = lax.axis_index("s") * NUM_CORES + lax.axis_index("c")
    base = wid * COLS_PER_W
    ids_bufs = (ids_v0, ids_v1)
    out_bufs = (out_v0, out_v1)
    in_sems = (in_sem0, in_sem1)
    out_sems = (out_sem0, out_sem1)

    pltpu.sync_copy(hop_hbm, table_v)
    pltpu.sync_copy(cut_hbm, cut_v)
    for i in range(64 // LANES):
        sl = pl.ds(i * LANES, LANES)
        table_v[sl] = table_v[sl] + cut_v[sl]

    def start_in(b):
        c0 = base + b * BLK_COLS
        return pltpu.async_copy(
            ids_hbm.at[:, pl.ds(c0, BLK_COLS)], ids_bufs[b % 2],
            in_sems[b % 2])

    def start_out(b):
        c0 = base + b * BLK_COLS
        return pltpu.async_copy(
            out_bufs[b % 2], out_hbm.at[:, pl.ds(c0, BLK_COLS)],
            out_sems[b % 2])

    in_dmas = {0: start_in(0)}
    out_dmas = {}
    for b in range(NBLK):
        if b + 1 < NBLK:
            in_dmas[b + 1] = start_in(b + 1)
        in_dmas[b].wait()
        if b >= 2:
            out_dmas[b - 2].wait()

        ids_b = ids_bufs[b % 2]
        out_b = out_bufs[b % 2]

        @plsc.parallel_loop(0, ROWS, 1, unroll=2)
        def gather_body(r):
            for j in range(VECS_PER_ROW):
                sl = pl.ds(j * LANES, LANES)
                out_b[r, sl] = plsc.load_gather(table_v, [ids_b[r, sl]])

        out_dmas[b] = start_out(b)

    out_dmas[NBLK - 2].wait()
    out_dmas[NBLK - 1].wait()


def kernel(ids_mat, hop_table, cut_off_table):
    return _sc_lookup(ids_mat.T, hop_table, cut_off_table).T


# speedup vs baseline: 1.9700x; 1.0703x over previous
import functools

import jax
import jax.numpy as jnp
from jax import lax
from jax.experimental import pallas as pl
from jax.experimental.pallas import tpu as pltpu
from jax.experimental.pallas import tpu_sc as plsc

ROWS = 200
COLS = 16384
NUM_CORES = 2
NUM_SUBCORES = 16
NW = NUM_CORES * NUM_SUBCORES
COLS_PER_W = COLS // NW
BLK_COLS = 128
NBLK = COLS_PER_W // BLK_COLS
LANES = 16
VECS_PER_ROW = BLK_COLS // LANES

_mesh = plsc.VectorSubcoreMesh(core_axis_name="c", subcore_axis_name="s")


@functools.partial(
    pl.kernel,
    mesh=_mesh,
    out_type=jax.ShapeDtypeStruct((ROWS, COLS), jnp.float32),
    compiler_params=pltpu.CompilerParams(
        needs_layout_passes=False,
        disable_bounds_checks=True,
        disable_semaphore_checks=True,
        skip_device_barrier=True,
    ),
    scratch_types=[
        pltpu.VMEM((64,), jnp.float32),
        pltpu.VMEM((64,), jnp.float32),
        pltpu.VMEM((ROWS, BLK_COLS), jnp.int32),
        pltpu.VMEM((ROWS, BLK_COLS), jnp.int32),
        pltpu.VMEM((ROWS, BLK_COLS), jnp.float32),
        pltpu.VMEM((ROWS, BLK_COLS), jnp.float32),
        pltpu.SemaphoreType.DMA,
        pltpu.SemaphoreType.DMA,
        pltpu.SemaphoreType.DMA,
        pltpu.SemaphoreType.DMA,
    ],
)
def _sc_lookup(ids_hbm, hop_hbm, cut_hbm, out_hbm,
               table_v, cut_v, ids_v0, ids_v1, out_v0, out_v1,
               in_sem0, in_sem1, out_sem0, out_sem1):
    wid = lax.axis_index("s") * NUM_CORES + lax.axis_index("c")
    base = wid * COLS_PER_W
    ids_bufs = (ids_v0, ids_v1)
    out_bufs = (out_v0, out_v1)
    in_sems = (in_sem0, in_sem1)
    out_sems = (out_sem0, out_sem1)

    def start_in(b):
        c0 = base + b * BLK_COLS
        return pltpu.async_copy(
            ids_hbm.at[:, pl.ds(c0, BLK_COLS)], ids_bufs[b % 2],
            in_sems[b % 2])

    def start_out(b):
        c0 = base + b * BLK_COLS
        return pltpu.async_copy(
            out_bufs[b % 2], out_hbm.at[:, pl.ds(c0, BLK_COLS)],
            out_sems[b % 2])

    in_dmas = {0: start_in(0)}

    pltpu.sync_copy(hop_hbm, table_v)
    pltpu.sync_copy(cut_hbm, cut_v)
    for i in range(64 // LANES):
        sl = pl.ds(i * LANES, LANES)
        table_v[sl] = table_v[sl] + cut_v[sl]

    out_dmas = {}
    for b in range(NBLK):
        if b + 1 < NBLK:
            in_dmas[b + 1] = start_in(b + 1)
        in_dmas[b].wait()
        if b >= 2:
            out_dmas[b - 2].wait()

        ids_b = ids_bufs[b % 2]
        out_b = out_bufs[b % 2]

        @plsc.parallel_loop(0, ROWS, 1, unroll=4)
        def gather_body(r):
            for j in range(VECS_PER_ROW):
                sl = pl.ds(j * LANES, LANES)
                out_b[r, sl] = plsc.load_gather(table_v, [ids_b[r, sl]])

        out_dmas[b] = start_out(b)

    out_dmas[NBLK - 2].wait()
    out_dmas[NBLK - 1].wait()


def kernel(ids_mat, hop_table, cut_off_table):
    return _sc_lookup(ids_mat.T, hop_table, cut_off_table).T
